# baseline (speedup 1.0000x reference)
import functools

import jax
import jax.numpy as jnp
from jax import lax
from jax.experimental import pallas as pl
from jax.experimental.pallas import tpu as pltpu
from jax.experimental.pallas import tpu_sc as plsc

L = 16
NSLOTS = 4


def _make_sc_kernel(B, H, W, C):
    mesh = plsc.VectorSubcoreMesh(core_axis_name="c", subcore_axis_name="s")
    n_vec = C // L

    @functools.partial(
        pl.kernel,
        mesh=mesh,
        out_type=jax.ShapeDtypeStruct((B, H, W, C), jnp.float32),
        scratch_types=[
            pltpu.VMEM((NSLOTS, W, C), jnp.float32),
            pltpu.VMEM((W, C), jnp.float32),
            pltpu.VMEM((C,), jnp.float32),
        ]
        + [pltpu.SemaphoreType.DMA] * (2 * NSLOTS),
    )
    def sc_kernel(x_hbm, row_hbm, col_hbm, out_hbm, buf, pos, rowv, *sems):
        isems = sems[:NSLOTS]
        osems = sems[NSLOTS:]
        h = lax.axis_index("s") * 2 + lax.axis_index("c")

        pltpu.sync_copy(col_hbm, pos)
        pltpu.sync_copy(row_hbm.at[h], rowv)

        def _pos_body(w, carry):
            for j in range(n_vec):
                sl = pl.ds(j * L, L)
                plsc.addupdate(pos.at[w, sl], rowv[sl])
            return carry

        lax.fori_loop(0, W, _pos_body, 0)

        for s in range(NSLOTS):
            pltpu.async_copy(x_hbm.at[s, h], buf.at[s], isems[s])

        def group(g, carry):
            for s in range(NSLOTS):
                b = NSLOTS * g + s
                pltpu.make_async_copy(
                    x_hbm.at[b, h], buf.at[s], isems[s]).wait()

                def _add_body(w, c2):
                    for j in range(n_vec):
                        sl = pl.ds(j * L, L)
                        plsc.addupdate(buf.at[s, w, sl], pos[w, sl])
                    return c2

                lax.fori_loop(0, W, _add_body, 0)

                pltpu.async_copy(buf.at[s], out_hbm.at[b, h], osems[s])

                sp = (s - 1) % NSLOTS
                bp = b - 1

                def retire_and_prefetch():
                    pltpu.make_async_copy(
                        buf.at[sp], out_hbm.at[bp, h], osems[sp]).wait()

                    @pl.when(bp + NSLOTS < B)
                    def _prefetch():
                        pltpu.async_copy(
                            x_hbm.at[bp + NSLOTS, h], buf.at[sp], isems[sp])

                if s == 0:
                    pl.when(g > 0)(retire_and_prefetch)
                else:
                    retire_and_prefetch()
            return carry

        lax.fori_loop(0, B // NSLOTS, group, 0)

        pltpu.make_async_copy(
            buf.at[NSLOTS - 1], out_hbm.at[B - 1, h],
            osems[NSLOTS - 1]).wait()

    return sc_kernel


def kernel(x, row_emb, col_emb):
    b, h, w, c = x.shape
    return _make_sc_kernel(b, h, w, c)(x, row_emb, col_emb)


# speedup vs baseline: 1.1002x; 1.1002x over previous
import functools

import jax
import jax.numpy as jnp
from jax import lax
from jax.experimental import pallas as pl
from jax.experimental.pallas import tpu as pltpu
from jax.experimental.pallas import tpu_sc as plsc

L = 16
NSLOTS = 4


def _make_sc_kernel(B, H, W, C):
    mesh = plsc.VectorSubcoreMesh(core_axis_name="c", subcore_axis_name="s")
    n_vec = C // L
    half = W // 2

    @functools.partial(
        pl.kernel,
        mesh=mesh,
        out_type=jax.ShapeDtypeStruct((B, H, W, C), jnp.float32),
        scratch_types=[
            pltpu.VMEM((NSLOTS, W, C), jnp.float32),
            pltpu.VMEM((W, C), jnp.float32),
            pltpu.VMEM((C,), jnp.float32),
        ]
        + [pltpu.SemaphoreType.DMA] * (2 * NSLOTS),
    )
    def sc_kernel(x_hbm, row_hbm, col_hbm, out_hbm, buf, pos, rowv, *sems):
        isems = sems[:NSLOTS]
        osems = sems[NSLOTS:]
        h = lax.axis_index("s") * 2 + lax.axis_index("c")

        pltpu.sync_copy(col_hbm, pos)
        pltpu.sync_copy(row_hbm.at[h], rowv)

        @plsc.parallel_loop(0, W, unroll=2)
        def _pos_body(w):
            for j in range(n_vec):
                sl = pl.ds(j * L, L)
                plsc.addupdate(pos.at[w, sl], rowv[sl])

        for s in range(NSLOTS):
            pltpu.async_copy(x_hbm.at[s, h], buf.at[s], isems[s])

        def group(g, carry):
            for s in range(NSLOTS):
                b = NSLOTS * g + s
                sp = (s - 1) % NSLOTS
                bp = b - 1

                pltpu.make_async_copy(
                    x_hbm.at[b, h], buf.at[s], isems[s]).wait()

                def retire_and_prefetch():
                    pltpu.make_async_copy(
                        buf.at[sp], out_hbm.at[bp, h], osems[sp]).wait()

                    @pl.when(bp + NSLOTS < B)
                    def _prefetch():
                        pltpu.async_copy(
                            x_hbm.at[bp + NSLOTS, h], buf.at[sp], isems[sp])

                if s == 0:
                    pl.when(g > 0)(retire_and_prefetch)
                else:
                    retire_and_prefetch()

                @plsc.parallel_loop(0, half, unroll=2)
                def _add_lo(w):
                    for j in range(n_vec):
                        sl = pl.ds(j * L, L)
                        plsc.addupdate(buf.at[s, w, sl], pos[w, sl])

                pltpu.async_copy(
                    buf.at[s, pl.ds(0, half)],
                    out_hbm.at[b, h, pl.ds(0, half)], osems[s])

                @plsc.parallel_loop(half, W, unroll=2)
                def _add_hi(w):
                    for j in range(n_vec):
                        sl = pl.ds(j * L, L)
                        plsc.addupdate(buf.at[s, w, sl], pos[w, sl])

                pltpu.async_copy(
                    buf.at[s, pl.ds(half, half)],
                    out_hbm.at[b, h, pl.ds(half, half)], osems[s])
            return carry

        lax.fori_loop(0, B // NSLOTS, group, 0)

        pltpu.make_async_copy(
            buf.at[NSLOTS - 1], out_hbm.at[B - 1, h],
            osems[NSLOTS - 1]).wait()

    return sc_kernel


def kernel(x, row_emb, col_emb):
    b, h, w, c = x.shape
    return _make_sc_kernel(b, h, w, c)(x, row_emb, col_emb)


# speedup vs baseline: 1.1639x; 1.0579x over previous
import functools

import jax
import jax.numpy as jnp
from jax import lax
from jax.experimental import pallas as pl
from jax.experimental.pallas import tpu as pltpu
from jax.experimental.pallas import tpu_sc as plsc

L = 16
NSLOTS = 4


def _make_sc_kernel(B, H, W, C):
    mesh = plsc.VectorSubcoreMesh(core_axis_name="c", subcore_axis_name="s")
    n_vec = C // L
    half = W // 2

    @functools.partial(
        pl.kernel,
        mesh=mesh,
        out_type=jax.ShapeDtypeStruct((B, H, W, C), jnp.float32),
        scratch_types=[
            pltpu.VMEM((NSLOTS, W, C), jnp.float32),
            pltpu.VMEM((W, C), jnp.float32),
            pltpu.VMEM((C,), jnp.float32),
        ]
        + [pltpu.SemaphoreType.DMA] * (2 * NSLOTS),
    )
    def sc_kernel(x_hbm, row_hbm, col_hbm, out_hbm, buf, pos, rowv, *sems):
        isems = sems[:NSLOTS]
        osems = sems[NSLOTS:]
        h = lax.axis_index("s") * 2 + lax.axis_index("c")

        pltpu.sync_copy(col_hbm, pos)
        pltpu.sync_copy(row_hbm.at[h], rowv)

        @plsc.parallel_loop(0, W, unroll=2)
        def _pos_body(w):
            for j in range(n_vec):
                sl = pl.ds(j * L, L)
                plsc.addupdate(pos.at[w, sl], rowv[sl])

        for s in range(NSLOTS):
            pltpu.async_copy(x_hbm.at[s, h], buf.at[s], isems[s])

        def group(g, carry):
            for s in range(NSLOTS):
                b = NSLOTS * g + s
                sp = (s - 1) % NSLOTS
                bp = b - 1

                pltpu.make_async_copy(
                    x_hbm.at[b, h], buf.at[s], isems[s]).wait()

                @plsc.parallel_loop(0, half, unroll=2)
                def _add_lo(w):
                    for j in range(n_vec):
                        sl = pl.ds(j * L, L)
                        plsc.addupdate(buf.at[s, w, sl], pos[w, sl])

                pltpu.async_copy(
                    buf.at[s, pl.ds(0, half)],
                    out_hbm.at[b, h, pl.ds(0, half)], osems[s])

                @plsc.parallel_loop(half, W, unroll=2)
                def _add_hi(w):
                    for j in range(n_vec):
                        sl = pl.ds(j * L, L)
                        plsc.addupdate(buf.at[s, w, sl], pos[w, sl])

                pltpu.async_copy(
                    buf.at[s, pl.ds(half, half)],
                    out_hbm.at[b, h, pl.ds(half, half)], osems[s])

                def retire_and_prefetch():
                    pltpu.make_async_copy(
                        buf.at[sp], out_hbm.at[bp, h], osems[sp]).wait()

                    @pl.when(bp + NSLOTS < B)
                    def _prefetch():
                        pltpu.async_copy(
                            x_hbm.at[bp + NSLOTS, h], buf.at[sp], isems[sp])

                if s == 0:
                    pl.when(g > 0)(retire_and_prefetch)
                else:
                    retire_and_prefetch()
            return carry

        lax.fori_loop(0, B // NSLOTS, group, 0)

        pltpu.make_async_copy(
            buf.at[NSLOTS - 1], out_hbm.at[B - 1, h],
            osems[NSLOTS - 1]).wait()

    return sc_kernel


def kernel(x, row_emb, col_emb):
    b, h, w, c = x.shape
    return _make_sc_kernel(b, h, w, c)(x, row_emb, col_emb)


# speedup vs baseline: 1.7293x; 1.4857x over previous
import functools

import jax
import jax.numpy as jnp
from jax import lax
from jax.experimental import pallas as pl
from jax.experimental.pallas import tpu as pltpu
from jax.experimental.pallas import tpu_sc as plsc

L = 16
NSLOTS = 4
DIST = 2


def _make_sc_kernel(B, H, W, C):
    mesh = plsc.VectorSubcoreMesh(core_axis_name="c", subcore_axis_name="s")
    n_vec = C // L

    @functools.partial(
        pl.kernel,
        mesh=mesh,
        out_type=jax.ShapeDtypeStruct((B, H, W, C), jnp.float32),
        scratch_types=[
            pltpu.VMEM((NSLOTS, W, C), jnp.float32),
            pltpu.VMEM((W, C), jnp.float32),
            pltpu.VMEM((C,), jnp.float32),
        ]
        + [pltpu.SemaphoreType.DMA] * (2 * NSLOTS),
    )
    def sc_kernel(x_hbm, row_hbm, col_hbm, out_hbm, buf, pos, rowv, *sems):
        isems = sems[:NSLOTS]
        osems = sems[NSLOTS:]
        h = lax.axis_index("s") * 2 + lax.axis_index("c")

        pltpu.sync_copy(col_hbm, pos)
        pltpu.sync_copy(row_hbm.at[h], rowv)

        @plsc.parallel_loop(0, W, unroll=2)
        def _pos_body(w):
            for j in range(n_vec):
                sl = pl.ds(j * L, L)
                plsc.addupdate(pos.at[w, sl], rowv[sl])

        for s in range(DIST):
            pltpu.async_copy(x_hbm.at[s, h], buf.at[s], isems[s])

        def group(g, carry):
            for s in range(NSLOTS):
                b = NSLOTS * g + s
                sp = (s - DIST) % NSLOTS
                bp = b - DIST

                pltpu.make_async_copy(
                    x_hbm.at[b, h], buf.at[s], isems[s]).wait()

                def retire():
                    pltpu.make_async_copy(
                        buf.at[sp], out_hbm.at[bp, h], osems[sp]).wait()

                if s < DIST:
                    pl.when(g > 0)(retire)
                else:
                    retire()

                @pl.when(b + DIST < B)
                def _prefetch():
                    pltpu.async_copy(
                        x_hbm.at[b + DIST, h], buf.at[sp], isems[sp])

                @plsc.parallel_loop(0, W, unroll=2)
                def _add_body(w):
                    for j in range(n_vec):
                        sl = pl.ds(j * L, L)
                        plsc.addupdate(buf.at[s, w, sl], pos[w, sl])

                pltpu.async_copy(buf.at[s], out_hbm.at[b, h], osems[s])
            return carry

        lax.fori_loop(0, B // NSLOTS, group, 0)

        for d in range(DIST, 0, -1):
            s = (B - d) % NSLOTS
            pltpu.make_async_copy(
                buf.at[s], out_hbm.at[B - d, h], osems[s]).wait()

    return sc_kernel


def kernel(x, row_emb, col_emb):
    b, h, w, c = x.shape
    return _make_sc_kernel(b, h, w, c)(x, row_emb, col_emb)


# speedup vs baseline: 1.7670x; 1.0218x over previous
import functools

import jax
import jax.numpy as jnp
from jax import lax
from jax.experimental import pallas as pl
from jax.experimental.pallas import tpu as pltpu
from jax.experimental.pallas import tpu_sc as plsc

L = 16
NSLOTS = 4
DIST = 2


def _make_sc_kernel(B, H, W, C):
    mesh = plsc.VectorSubcoreMesh(core_axis_name="c", subcore_axis_name="s")
    n_vec = C // L

    @functools.partial(
        pl.kernel,
        mesh=mesh,
        out_type=jax.ShapeDtypeStruct((B, H, W, C), jnp.float32),
        scratch_types=[
            pltpu.VMEM((NSLOTS, W, C), jnp.float32),
            pltpu.VMEM((W, C), jnp.float32),
            pltpu.VMEM((C,), jnp.float32),
        ]
        + [pltpu.SemaphoreType.DMA] * (2 * NSLOTS),
    )
    def sc_kernel(x_hbm, row_hbm, col_hbm, out_hbm, buf, pos, rowv, *sems):
        isems = sems[:NSLOTS]
        osems = sems[NSLOTS:]
        h = lax.axis_index("s") * 2 + lax.axis_index("c")

        pltpu.sync_copy(col_hbm, pos)
        pltpu.sync_copy(row_hbm.at[h], rowv)

        @plsc.parallel_loop(0, W, unroll=2)
        def _pos_body(w):
            for j in range(n_vec):
                sl = pl.ds(j * L, L)
                plsc.addupdate(pos.at[w, sl], rowv[sl])

        for s in range(DIST):
            pltpu.async_copy(x_hbm.at[s, h], buf.at[s], isems[s])

        def group(g, carry):
            for s in (0, 2):
                b = NSLOTS * g + s
                s0, s1 = s, s + 1
                p0, p1 = (s + 2) % NSLOTS, (s + 3) % NSLOTS

                pltpu.make_async_copy(
                    x_hbm.at[b, h], buf.at[s0], isems[s0]).wait()
                pltpu.make_async_copy(
                    x_hbm.at[b + 1, h], buf.at[s1], isems[s1]).wait()

                def retire():
                    pltpu.make_async_copy(
                        buf.at[p0], out_hbm.at[b - 2, h], osems[p0]).wait()
                    pltpu.make_async_copy(
                        buf.at[p1], out_hbm.at[b - 1, h], osems[p1]).wait()

                if s == 0:
                    pl.when(g > 0)(retire)
                else:
                    retire()

                @pl.when(b + 2 < B)
                def _prefetch():
                    pltpu.async_copy(
                        x_hbm.at[b + 2, h], buf.at[p0], isems[p0])
                    pltpu.async_copy(
                        x_hbm.at[b + 3, h], buf.at[p1], isems[p1])

                @plsc.parallel_loop(0, W, unroll=2)
                def _add_body(w):
                    for j in range(n_vec):
                        sl = pl.ds(j * L, L)
                        pv = pos[w, sl]
                        plsc.addupdate(buf.at[s0, w, sl], pv)
                        plsc.addupdate(buf.at[s1, w, sl], pv)

                pltpu.async_copy(buf.at[s0], out_hbm.at[b, h], osems[s0])
                pltpu.async_copy(buf.at[s1], out_hbm.at[b + 1, h], osems[s1])
            return carry

        lax.fori_loop(0, B // NSLOTS, group, 0)

        for d in range(DIST, 0, -1):
            s = (B - d) % NSLOTS
            pltpu.make_async_copy(
                buf.at[s], out_hbm.at[B - d, h], osems[s]).wait()

    return sc_kernel


def kernel(x, row_emb, col_emb):
    b, h, w, c = x.shape
    return _make_sc_kernel(b, h, w, c)(x, row_emb, col_emb)
